# baseline (speedup 1.0000x reference)
import functools

import jax
import jax.numpy as jnp
from jax import lax
from jax.experimental import pallas as pl
from jax.experimental.pallas import tpu as pltpu
from jax.experimental.pallas import tpu_sc as plsc

B = 16384
D = 128
L = 16
NC, NS = 2, 16
NW = NC * NS
BPW = B // NW
CH = 128
NCH = BPW // CH


def _mf_body(ui_hbm, ii_hbm, ue_hbm, ie_hbm, ub_hbm, ib_hbm, gb_hbm,
             out_hbm,
             ui_v, ii_v, u_rows, i_rows, ub_v, ib_v, gb_v, out_v,
             sem_ui, sem_ii, sem_u, sem_i, sem_ub, sem_ib,
             sem_su0, sem_su1, sem_su2, sem_su3,
             sem_si0, sem_si1, sem_si2, sem_si3):
    wid = lax.axis_index("s") * NC + lax.axis_index("c")
    base = wid * BPW

    stage = (pltpu.make_async_copy(ui_hbm.at[wid], ui_v, sem_ui),
             pltpu.make_async_copy(ii_hbm.at[wid], ii_v, sem_ii),
             pltpu.make_async_copy(gb_hbm, gb_v, sem_ub))
    for cp in stage:
        cp.start()
    for cp in stage:
        cp.wait()
    lane = lax.iota(jnp.int32, L)
    zero16 = jnp.zeros((L,), jnp.int32)
    gbv = plsc.load_gather(gb_v, [zero16])
    last_lane = lane == (L - 1)

    def row_descs(c, buf):
        return (
            pltpu.make_async_copy(ue_hbm.at[ui_v.at[c]], u_rows.at[buf],
                                  sem_u),
            pltpu.make_async_copy(ie_hbm.at[ii_v.at[c]], i_rows.at[buf],
                                  sem_i),
        )

    def bias_descs(c, buf):
        return (
            pltpu.make_async_copy(ub_hbm.at[ui_v.at[c]], ub_v.at[buf],
                                  sem_ub),
            pltpu.make_async_copy(ib_hbm.at[ii_v.at[c]], ib_v.at[buf],
                                  sem_ib),
        )

    def row_loop(u_c, i_c, out_base, lo, hi):
        @plsc.parallel_loop(lo, hi, unroll=4)
        def _row(r):
            parts = [u_c[r, pl.ds(j * L, L)] * i_c[r, pl.ds(j * L, L)]
                     for j in range(D // L)]
            s = ((parts[0] + parts[1]) + (parts[2] + parts[3])) + \
                ((parts[4] + parts[5]) + (parts[6] + parts[7]))
            cs = plsc.cumsum(s)
            plsc.store_compressed(out_v.at[pl.ds(out_base + r, L)], cs,
                                  mask=last_lane)

    def bias_pass(buf, out_base):
        @plsc.parallel_loop(0, CH // L)
        def _bias(g):
            pos = out_base + g * L
            o = (out_v[pl.ds(pos, L)] + gbv
                 + ub_v[buf, pl.ds(g * L, L)] + ib_v[buf, pl.ds(g * L, L)])
            out_v[pl.ds(pos, L)] = o

    SUB = 32
    NSUB = CH // SUB
    sem_su = (sem_su0, sem_su1, sem_su2, sem_su3)
    sem_si = (sem_si0, sem_si1, sem_si2, sem_si3)
    subs = [
        (pltpu.make_async_copy(ue_hbm.at[ui_v.at[0, pl.ds(s * SUB, SUB)]],
                               u_rows.at[0, pl.ds(s * SUB, SUB)], sem_su[s]),
         pltpu.make_async_copy(ie_hbm.at[ii_v.at[0, pl.ds(s * SUB, SUB)]],
                               i_rows.at[0, pl.ds(s * SUB, SUB)], sem_si[s]))
        for s in range(NSUB)
    ]
    for pair in subs:
        for cp in pair:
            cp.start()
    for cp in bias_descs(0, 0) + row_descs(1, 1) + bias_descs(1, 1):
        cp.start()

    for s in range(NSUB):
        for cp in subs[s]:
            cp.wait()
        row_loop(u_rows.at[0], i_rows.at[0], 0, s * SUB, (s + 1) * SUB)
    for cp in bias_descs(0, 0):
        cp.wait()
    bias_pass(0, 0)

    @pl.loop(1, NCH)
    def _chunk(c):
        buf = lax.rem(c, 2)
        for cp in row_descs(c, buf):
            cp.wait()

        @pl.when(c < NCH - 1)
        def _start_next():
            for cp in row_descs(c + 1, 1 - buf) + bias_descs(c + 1, 1 - buf):
                cp.start()

        row_loop(u_rows.at[buf], i_rows.at[buf], c * CH, 0, CH)

        for cp in bias_descs(c, buf):
            cp.wait()
        bias_pass(buf, c * CH)

    pltpu.sync_copy(out_v.at[pl.ds(0, BPW)], out_hbm.at[pl.ds(base, BPW)])


@functools.partial(jax.jit, static_argnums=())
def _mf_call(ui3, ii3, ue, ie, ub1, ib1, gb):
    mesh = plsc.VectorSubcoreMesh(core_axis_name="c", subcore_axis_name="s",
                                  num_cores=NC, num_subcores=NS)
    f = pl.kernel(
        _mf_body,
        out_type=jax.ShapeDtypeStruct((B,), jnp.float32),
        mesh=mesh,
        compiler_params=pltpu.CompilerParams(needs_layout_passes=False,
                                             disable_bounds_checks=True),
        scratch_types=[
            pltpu.VMEM((NCH, CH), jnp.int32),
            pltpu.VMEM((NCH, CH), jnp.int32),
            pltpu.VMEM((2, CH, D), jnp.float32),
            pltpu.VMEM((2, CH, D), jnp.float32),
            pltpu.VMEM((2, CH), jnp.float32),
            pltpu.VMEM((2, CH), jnp.float32),
            pltpu.VMEM((1,), jnp.float32),
            pltpu.VMEM((BPW + L,), jnp.float32),
            pltpu.SemaphoreType.DMA,
            pltpu.SemaphoreType.DMA,
            pltpu.SemaphoreType.DMA,
            pltpu.SemaphoreType.DMA,
            pltpu.SemaphoreType.DMA,
            pltpu.SemaphoreType.DMA,
            pltpu.SemaphoreType.DMA,
            pltpu.SemaphoreType.DMA,
            pltpu.SemaphoreType.DMA,
            pltpu.SemaphoreType.DMA,
            pltpu.SemaphoreType.DMA,
            pltpu.SemaphoreType.DMA,
            pltpu.SemaphoreType.DMA,
            pltpu.SemaphoreType.DMA,
        ],
        name="mf_kernel",
    )
    return f(ui3, ii3, ue, ie, ub1, ib1, gb)


def kernel(user_indices, item_indices, user_embedding, item_embedding,
           user_bias, item_bias, global_bias):
    ui3 = user_indices.reshape(NW, NCH, CH)
    ii3 = item_indices.reshape(NW, NCH, CH)
    ub1 = user_bias.reshape(-1)
    ib1 = item_bias.reshape(-1)
    return _mf_call(ui3, ii3, user_embedding, item_embedding, ub1, ib1,
                    global_bias)


# speedup vs baseline: 1.0013x; 1.0013x over previous
import functools

import jax
import jax.numpy as jnp
from jax import lax
from jax.experimental import pallas as pl
from jax.experimental.pallas import tpu as pltpu
from jax.experimental.pallas import tpu_sc as plsc

B = 16384
D = 128
L = 16
NC, NS = 2, 16
NW = NC * NS
BPW = B // NW
CH = 128
NCH = BPW // CH


def _mf_body(ui_hbm, ii_hbm, ue_hbm, ie_hbm, ub_hbm, ib_hbm, gb_hbm,
             out_hbm,
             ui_v, ii_v, u_rows, i_rows, ub_v, ib_v, gb_v, out_v,
             sem_ui, sem_ii, sem_u, sem_i, sem_ub, sem_ib):
    wid = lax.axis_index("s") * NC + lax.axis_index("c")
    base = wid * BPW

    stage = (pltpu.make_async_copy(ui_hbm.at[wid], ui_v, sem_ui),
             pltpu.make_async_copy(ii_hbm.at[wid], ii_v, sem_ii),
             pltpu.make_async_copy(gb_hbm, gb_v, sem_ub))
    for cp in stage:
        cp.start()
    for cp in stage:
        cp.wait()
    lane = lax.iota(jnp.int32, L)
    zero16 = jnp.zeros((L,), jnp.int32)
    gbv = plsc.load_gather(gb_v, [zero16])
    last_lane = lane == (L - 1)

    def row_descs(c, buf):
        return (
            pltpu.make_async_copy(ue_hbm.at[ui_v.at[c]], u_rows.at[buf],
                                  sem_u),
            pltpu.make_async_copy(ie_hbm.at[ii_v.at[c]], i_rows.at[buf],
                                  sem_i),
        )

    def bias_descs(c, buf):
        return (
            pltpu.make_async_copy(ub_hbm.at[ui_v.at[c]], ub_v.at[buf],
                                  sem_ub),
            pltpu.make_async_copy(ib_hbm.at[ii_v.at[c]], ib_v.at[buf],
                                  sem_ib),
        )

    for cp in row_descs(0, 0) + bias_descs(0, 0):
        cp.start()

    for c in range(NCH):
        buf = c % 2
        for cp in row_descs(c, buf):
            cp.wait()
        if c < NCH - 1:
            for cp in row_descs(c + 1, 1 - buf) + bias_descs(c + 1, 1 - buf):
                cp.start()

        u_c = u_rows.at[buf]
        i_c = i_rows.at[buf]

        @plsc.parallel_loop(0, CH, unroll=4)
        def _row(r):
            parts = [u_c[r, pl.ds(j * L, L)] * i_c[r, pl.ds(j * L, L)]
                     for j in range(D // L)]
            s = ((parts[0] + parts[1]) + (parts[2] + parts[3])) + \
                ((parts[4] + parts[5]) + (parts[6] + parts[7]))
            cs = plsc.cumsum(s)
            plsc.store_compressed(out_v.at[pl.ds(c * CH + r, L)], cs,
                                  mask=last_lane)

        for cp in bias_descs(c, buf):
            cp.wait()

        @plsc.parallel_loop(0, CH // L)
        def _bias(g):
            pos = c * CH + g * L
            o = (out_v[pl.ds(pos, L)] + gbv
                 + ub_v[buf, pl.ds(g * L, L)] + ib_v[buf, pl.ds(g * L, L)])
            out_v[pl.ds(pos, L)] = o

    pltpu.sync_copy(out_v.at[pl.ds(0, BPW)], out_hbm.at[pl.ds(base, BPW)])


@functools.partial(jax.jit, static_argnums=())
def _mf_call(ui3, ii3, ue, ie, ub1, ib1, gb):
    mesh = plsc.VectorSubcoreMesh(core_axis_name="c", subcore_axis_name="s",
                                  num_cores=NC, num_subcores=NS)
    f = pl.kernel(
        _mf_body,
        out_type=jax.ShapeDtypeStruct((B,), jnp.float32),
        mesh=mesh,
        compiler_params=pltpu.CompilerParams(needs_layout_passes=False,
                                             disable_bounds_checks=True),
        scratch_types=[
            pltpu.VMEM((NCH, CH), jnp.int32),
            pltpu.VMEM((NCH, CH), jnp.int32),
            pltpu.VMEM((2, CH, D), jnp.float32),
            pltpu.VMEM((2, CH, D), jnp.float32),
            pltpu.VMEM((2, CH), jnp.float32),
            pltpu.VMEM((2, CH), jnp.float32),
            pltpu.VMEM((1,), jnp.float32),
            pltpu.VMEM((BPW + L,), jnp.float32),
            pltpu.SemaphoreType.DMA,
            pltpu.SemaphoreType.DMA,
            pltpu.SemaphoreType.DMA,
            pltpu.SemaphoreType.DMA,
            pltpu.SemaphoreType.DMA,
            pltpu.SemaphoreType.DMA,
        ],
        name="mf_kernel",
    )
    return f(ui3, ii3, ue, ie, ub1, ib1, gb)


def kernel(user_indices, item_indices, user_embedding, item_embedding,
           user_bias, item_bias, global_bias):
    ui3 = user_indices.reshape(NW, NCH, CH)
    ii3 = item_indices.reshape(NW, NCH, CH)
    ub1 = user_bias.reshape(-1)
    ib1 = item_bias.reshape(-1)
    return _mf_call(ui3, ii3, user_embedding, item_embedding, ub1, ib1,
                    global_bias)


# speedup vs baseline: 1.0179x; 1.0166x over previous
import functools

import jax
import jax.numpy as jnp
from jax import lax
from jax.experimental import pallas as pl
from jax.experimental.pallas import tpu as pltpu
from jax.experimental.pallas import tpu_sc as plsc

B = 16384
D = 128
L = 16
NC, NS = 2, 16
NW = NC * NS
BPW = B // NW
CH = 128
NCH = BPW // CH


def _mf_body(ui_hbm, ii_hbm, ue_hbm, ie_hbm, ub_hbm, ib_hbm, gb_hbm,
             out_hbm,
             ui_v, ii_v, u_rows, i_rows, ub_v, ib_v, gb_v, out_v,
             sem_ui, sem_ii, sem_u, sem_i, sem_ub, sem_ib):
    wid = lax.axis_index("s") * NC + lax.axis_index("c")
    base = wid * BPW

    stage = (pltpu.make_async_copy(ui_hbm.at[wid], ui_v, sem_ui),
             pltpu.make_async_copy(ii_hbm.at[wid], ii_v, sem_ii),
             pltpu.make_async_copy(gb_hbm, gb_v, sem_ub))
    for cp in stage:
        cp.start()
    for cp in stage:
        cp.wait()
    lane = lax.iota(jnp.int32, L)
    zero16 = jnp.zeros((L,), jnp.int32)
    gbv = plsc.load_gather(gb_v, [zero16])
    last_lane = lane == (L - 1)

    def row_descs(c, buf):
        return (
            pltpu.make_async_copy(ue_hbm.at[ui_v.at[c]], u_rows.at[buf],
                                  sem_u),
            pltpu.make_async_copy(ie_hbm.at[ii_v.at[c]], i_rows.at[buf],
                                  sem_i),
        )

    def bias_descs(c, buf):
        return (
            pltpu.make_async_copy(ub_hbm.at[ui_v.at[c]], ub_v.at[buf],
                                  sem_ub),
            pltpu.make_async_copy(ib_hbm.at[ii_v.at[c]], ib_v.at[buf],
                                  sem_ib),
        )

    for cp in row_descs(0, 0) + bias_descs(0, 0):
        cp.start()

    @pl.loop(0, NCH)
    def _chunk(c):
        buf = lax.rem(c, 2)
        for cp in row_descs(c, buf):
            cp.wait()

        @pl.when(c < NCH - 1)
        def _start_next():
            for cp in row_descs(c + 1, 1 - buf) + bias_descs(c + 1, 1 - buf):
                cp.start()

        u_c = u_rows.at[buf]
        i_c = i_rows.at[buf]

        @plsc.parallel_loop(0, CH, unroll=4)
        def _row(r):
            parts = [u_c[r, pl.ds(j * L, L)] * i_c[r, pl.ds(j * L, L)]
                     for j in range(D // L)]
            s = ((parts[0] + parts[1]) + (parts[2] + parts[3])) + \
                ((parts[4] + parts[5]) + (parts[6] + parts[7]))
            cs = plsc.cumsum(s)
            plsc.store_compressed(out_v.at[pl.ds(c * CH + r, L)], cs,
                                  mask=last_lane)

        for cp in bias_descs(c, buf):
            cp.wait()

        @plsc.parallel_loop(0, CH // L)
        def _bias(g):
            pos = c * CH + g * L
            o = (out_v[pl.ds(pos, L)] + gbv
                 + ub_v[buf, pl.ds(g * L, L)] + ib_v[buf, pl.ds(g * L, L)])
            out_v[pl.ds(pos, L)] = o

    pltpu.sync_copy(out_v.at[pl.ds(0, BPW)], out_hbm.at[pl.ds(base, BPW)])


@functools.partial(jax.jit, static_argnums=())
def _mf_call(ui3, ii3, ue, ie, ub1, ib1, gb):
    mesh = plsc.VectorSubcoreMesh(core_axis_name="c", subcore_axis_name="s",
                                  num_cores=NC, num_subcores=NS)
    f = pl.kernel(
        _mf_body,
        out_type=jax.ShapeDtypeStruct((B,), jnp.float32),
        mesh=mesh,
        compiler_params=pltpu.CompilerParams(needs_layout_passes=False,
                                             disable_bounds_checks=True),
        scratch_types=[
            pltpu.VMEM((NCH, CH), jnp.int32),
            pltpu.VMEM((NCH, CH), jnp.int32),
            pltpu.VMEM((2, CH, D), jnp.float32),
            pltpu.VMEM((2, CH, D), jnp.float32),
            pltpu.VMEM((2, CH), jnp.float32),
            pltpu.VMEM((2, CH), jnp.float32),
            pltpu.VMEM((1,), jnp.float32),
            pltpu.VMEM((BPW + L,), jnp.float32),
            pltpu.SemaphoreType.DMA,
            pltpu.SemaphoreType.DMA,
            pltpu.SemaphoreType.DMA,
            pltpu.SemaphoreType.DMA,
            pltpu.SemaphoreType.DMA,
            pltpu.SemaphoreType.DMA,
        ],
        name="mf_kernel",
    )
    return f(ui3, ii3, ue, ie, ub1, ib1, gb)


def kernel(user_indices, item_indices, user_embedding, item_embedding,
           user_bias, item_bias, global_bias):
    ui3 = user_indices.reshape(NW, NCH, CH)
    ii3 = item_indices.reshape(NW, NCH, CH)
    ub1 = user_bias.reshape(-1)
    ib1 = item_bias.reshape(-1)
    return _mf_call(ui3, ii3, user_embedding, item_embedding, ub1, ib1,
                    global_bias)
